# baseline (speedup 1.0000x reference)
import jax
import jax.numpy as jnp
from jax.experimental import pallas as pl
from jax.experimental.pallas import tpu as pltpu

EPS = 1e-5
VMEM_LIMIT = 64 * 1024 * 1024
NB = 2


def _cparams():
    return pltpu.CompilerParams(dimension_semantics=("parallel",),
                                vmem_limit_bytes=VMEM_LIMIT)


def _im2col_dot(pad_ref, w_ref, NBk, H, W, Cin):
    cols = [
        pad_ref[:, dy:dy + H, dx:dx + W, :].reshape(NBk * H * W, Cin)
        for dy in range(3) for dx in range(3)
    ]
    patches = jnp.concatenate(cols, axis=-1)
    return jnp.dot(patches, w_ref[...], preferred_element_type=jnp.float32)


def _channel_stats(acc):
    both = jnp.concatenate([acc, acc * acc], axis=-1)
    s = jnp.sum(both, axis=0, keepdims=True)
    return s.reshape(1, 1, s.shape[-1])


def _conv1_kernel(x_ref, w_ref, z_ref, st_ref, pad_ref):
    NBk, H, W, Cin = x_ref.shape
    Cout = w_ref.shape[1]
    pad_ref[...] = jnp.zeros_like(pad_ref)
    pad_ref[:, 1:H + 1, 1:W + 1, :] = x_ref[...]
    acc = _im2col_dot(pad_ref, w_ref, NBk, H, W, Cin)
    z_ref[...] = acc.astype(jnp.bfloat16).reshape(NBk, H, W * Cout)
    st_ref[...] = _channel_stats(acc)


def _bnrelu_conv2_kernel(z1_ref, ss_ref, w_ref, z2_ref, st_ref, pad_ref):
    NBk, H, WC = z1_ref.shape
    Cmid = pad_ref.shape[3]
    W = WC // Cmid
    Cout = w_ref.shape[1]
    a1 = jnp.maximum(z1_ref[...] * ss_ref[0] + ss_ref[1], 0.0)
    pad_ref[...] = jnp.zeros_like(pad_ref)
    pad_ref[:, 1:H + 1, 1:W + 1, :] = a1.astype(jnp.bfloat16).reshape(NBk, H, W, Cmid)
    acc = _im2col_dot(pad_ref, w_ref, NBk, H, W, Cmid)
    z2_ref[...] = acc.astype(jnp.bfloat16).reshape(NBk, H, W * Cout)
    st_ref[...] = _channel_stats(acc)


def _bnrelu_kernel(z_ref, ss_ref, o_ref):
    o_ref[...] = jnp.maximum(z_ref[...] * ss_ref[0] + ss_ref[1], 0.0)


def _conv1(x_nhwc, w_slab):
    N, H, W, Cin = x_nhwc.shape
    Cout = w_slab.shape[1]
    return pl.pallas_call(
        _conv1_kernel,
        out_shape=(jax.ShapeDtypeStruct((N, H, W * Cout), jnp.bfloat16),
                   jax.ShapeDtypeStruct((N // NB, 1, 2 * Cout), jnp.float32)),
        grid=(N // NB,),
        in_specs=[
            pl.BlockSpec((NB, H, W, Cin), lambda n: (n, 0, 0, 0)),
            pl.BlockSpec((9 * Cin, Cout), lambda n: (0, 0)),
        ],
        out_specs=(
            pl.BlockSpec((NB, H, W * Cout), lambda n: (n, 0, 0)),
            pl.BlockSpec((1, 1, 2 * Cout), lambda n: (n, 0, 0)),
        ),
        scratch_shapes=[pltpu.VMEM((NB, H + 2, W + 2, Cin), jnp.bfloat16)],
        compiler_params=_cparams(),
    )(x_nhwc, w_slab)


def _bnrelu_conv2(z1_slab, ss1, w_slab, cmid):
    N, H, WC = z1_slab.shape
    W = WC // cmid
    Cout = w_slab.shape[1]
    return pl.pallas_call(
        _bnrelu_conv2_kernel,
        out_shape=(jax.ShapeDtypeStruct((N, H, W * Cout), jnp.bfloat16),
                   jax.ShapeDtypeStruct((N // NB, 1, 2 * Cout), jnp.float32)),
        grid=(N // NB,),
        in_specs=[
            pl.BlockSpec((NB, H, WC), lambda n: (n, 0, 0)),
            pl.BlockSpec((2, WC), lambda n: (0, 0)),
            pl.BlockSpec((9 * cmid, Cout), lambda n: (0, 0)),
        ],
        out_specs=(
            pl.BlockSpec((NB, H, W * Cout), lambda n: (n, 0, 0)),
            pl.BlockSpec((1, 1, 2 * Cout), lambda n: (n, 0, 0)),
        ),
        scratch_shapes=[pltpu.VMEM((NB, H + 2, W + 2, cmid), jnp.bfloat16)],
        compiler_params=_cparams(),
    )(z1_slab, ss1, w_slab)


def _bnrelu(z_slab, ss):
    N, H, WC = z_slab.shape
    return pl.pallas_call(
        _bnrelu_kernel,
        out_shape=jax.ShapeDtypeStruct((N, H, WC), jnp.float32),
        grid=(N // NB,),
        in_specs=[
            pl.BlockSpec((NB, H, WC), lambda n: (n, 0, 0)),
            pl.BlockSpec((2, WC), lambda n: (0, 0)),
        ],
        out_specs=pl.BlockSpec((NB, H, WC), lambda n: (n, 0, 0)),
        compiler_params=_cparams(),
    )(z_slab, ss)


def _bn_scale_shift(stats, count, gamma, beta, W):
    C = gamma.shape[0]
    s = jnp.sum(stats.reshape(-1, 2 * C), axis=0)
    mean = s[:C] / count
    var = s[C:] / count - mean * mean
    scale = gamma * jax.lax.rsqrt(var + EPS)
    shift = beta - mean * scale
    return jnp.stack([jnp.tile(scale, W), jnp.tile(shift, W)], axis=0)


def kernel(x, w1, w2, gamma1, beta1, gamma2, beta2):
    xt = jnp.transpose(x, (0, 2, 3, 1)).astype(jnp.bfloat16)
    N, H, W, Cin = xt.shape
    mid_ch = gamma1.shape[0]
    out_ch = gamma2.shape[0]
    w1s = w1.reshape(9 * Cin, mid_ch).astype(jnp.bfloat16)
    w2s = w2.reshape(9 * mid_ch, out_ch).astype(jnp.bfloat16)
    count = float(N * H * W)

    z1, st1 = _conv1(xt, w1s)
    ss1 = _bn_scale_shift(st1, count, gamma1, beta1, W)
    z2, st2 = _bnrelu_conv2(z1, ss1, w2s, mid_ch)
    ss2 = _bn_scale_shift(st2, count, gamma2, beta2, W)
    a2 = _bnrelu(z2, ss2)

    return jnp.transpose(a2.reshape(N, H, W, out_ch), (0, 3, 1, 2))
